# Initial kernel scaffold; baseline (speedup 1.0000x reference)
import jax
import jax.numpy as jnp
from jax.experimental import pallas as pl


def kernel(inputs, Wg, bg, We, be):
    raise NotImplementedError("write your pallas kernel here")


# speedup vs baseline: 2.1767x; 2.1767x over previous
import jax
import jax.numpy as jnp
from jax.experimental import pallas as pl
from jax.experimental.pallas import tpu as pltpu

D_MODEL = 768
N_EXPERTS = 16
TOP_K = 2


def _moe_dense_kernel(x_ref, wg_ref, bg_ref, we_ref, be_ref, out_ref, w_scr):
    e = pl.program_id(0)

    @pl.when(e == 0)
    def _route():
        logits = (
            jnp.dot(x_ref[...], wg_ref[...], preferred_element_type=jnp.float32)
            + bg_ref[...]
        )
        col = jax.lax.broadcasted_iota(jnp.int32, logits.shape, 1)
        big = jnp.int32(N_EXPERTS)
        m1 = jnp.max(logits, axis=1, keepdims=True)
        idx1 = jnp.min(jnp.where(logits == m1, col, big), axis=1, keepdims=True)
        sel1 = col == idx1
        masked = jnp.where(sel1, -jnp.inf, logits)
        m2 = jnp.max(masked, axis=1, keepdims=True)
        idx2 = jnp.min(jnp.where(masked == m2, col, big), axis=1, keepdims=True)
        sel2 = col == idx2
        w1 = 1.0 / (1.0 + jnp.exp(m2 - m1))
        w2 = 1.0 - w1
        w_scr[...] = jnp.where(sel1, w1, 0.0) + jnp.where(sel2, w2, 0.0)

    w_all = w_scr[...]
    ecol = jax.lax.broadcasted_iota(jnp.int32, w_all.shape, 1)
    w_e = jnp.sum(jnp.where(ecol == e, w_all, 0.0), axis=1, keepdims=True)
    contrib = w_e * (
        jnp.dot(x_ref[...], we_ref[0], preferred_element_type=jnp.float32)
        + be_ref[0]
    )

    @pl.when(e == 0)
    def _init():
        out_ref[...] = contrib

    @pl.when(e > 0)
    def _acc():
        out_ref[...] += contrib


def kernel(inputs, Wg, bg, We, be):
    T = inputs.shape[0] * inputs.shape[1]
    x2 = inputs.reshape(T, D_MODEL)
    bg2 = bg.reshape(1, N_EXPERTS)
    be2 = be.reshape(N_EXPERTS, 1, D_MODEL)

    out = pl.pallas_call(
        _moe_dense_kernel,
        grid=(N_EXPERTS,),
        in_specs=[
            pl.BlockSpec((T, D_MODEL), lambda e: (0, 0)),
            pl.BlockSpec((D_MODEL, N_EXPERTS), lambda e: (0, 0)),
            pl.BlockSpec((1, N_EXPERTS), lambda e: (0, 0)),
            pl.BlockSpec((1, D_MODEL, D_MODEL), lambda e: (e, 0, 0)),
            pl.BlockSpec((1, 1, D_MODEL), lambda e: (e, 0, 0)),
        ],
        out_specs=pl.BlockSpec((T, D_MODEL), lambda e: (0, 0)),
        out_shape=jax.ShapeDtypeStruct((T, D_MODEL), jnp.float32),
        scratch_shapes=[pltpu.VMEM((T, N_EXPERTS), jnp.float32)],
    )(x2, Wg, bg2, We, be2)
    return out.reshape(inputs.shape)
